# baseline (speedup 1.0000x reference)
import jax
import jax.numpy as jnp
import numpy as np
from jax.experimental import pallas as pl
from jax.experimental.pallas import tpu as pltpu

NBINS = 15
N_ROWS = 16384
N_COLS = 1000
BLOCK_ROWS = 1024
GRID = N_ROWS // BLOCK_ROWS

_BOUNDS = np.linspace(0.0, 1.0, NBINS + 1).astype(np.float32)
_LOWER = _BOUNDS[:-1].reshape(1, NBINS)
_UPPER = _BOUNDS[1:].reshape(1, NBINS)


def _rows_kernel(logits_ref, targs_ref, conf_ref, acc_ref):
    x = logits_ref[...]
    targ = targs_ref[0, 0, :]
    col = jax.lax.broadcasted_iota(jnp.int32, x.shape, 1)
    hit = col == targ[:, None]
    m = jnp.max(x, axis=1, keepdims=True)
    x_t = jnp.max(jnp.where(hit, x, -jnp.inf), axis=1)
    s = jnp.sum(jnp.exp(x - m), axis=1)
    conf_ref[0, 0, :] = 1.0 / s
    acc_ref[0, 0, :] = (x_t == m[:, 0]).astype(jnp.float32)


def _ece_kernel(conf_ref, acc_ref, out_ref):
    conf = conf_ref[...]
    acc = acc_ref[...]
    ece = jnp.float32(0.0)
    for b in range(NBINS):
        lb = float(_BOUNDS[b])
        ub = float(_BOUNDS[b + 1])
        mask = ((conf > lb) & (conf <= ub)).astype(jnp.float32)
        cnt = jnp.sum(mask)
        conf_s = jnp.sum(mask * conf)
        acc_s = jnp.sum(mask * acc)
        safe = jnp.maximum(cnt, 1.0)
        gap = jnp.abs(conf_s / safe - acc_s / safe) * (cnt * (1.0 / N_ROWS))
        ece = ece + jnp.where(cnt > 0.0, gap, 0.0)
    out_ref[...] = jnp.full((1, 1), ece)


@jax.jit
def kernel(logits, targs):
    targs3 = targs.reshape(GRID, 1, BLOCK_ROWS)
    conf, acc = pl.pallas_call(
        _rows_kernel,
        grid=(GRID,),
        in_specs=[
            pl.BlockSpec((BLOCK_ROWS, N_COLS), lambda i: (i, 0)),
            pl.BlockSpec((1, 1, BLOCK_ROWS), lambda i: (i, 0, 0)),
        ],
        out_specs=[
            pl.BlockSpec((1, 1, BLOCK_ROWS), lambda i: (i, 0, 0)),
            pl.BlockSpec((1, 1, BLOCK_ROWS), lambda i: (i, 0, 0)),
        ],
        out_shape=[
            jax.ShapeDtypeStruct((GRID, 1, BLOCK_ROWS), jnp.float32),
            jax.ShapeDtypeStruct((GRID, 1, BLOCK_ROWS), jnp.float32),
        ],
        compiler_params=pltpu.CompilerParams(
            dimension_semantics=("parallel",)
        ),
    )(logits, targs3)
    conf2 = conf.reshape(128, 128)
    acc2 = acc.reshape(128, 128)
    out = pl.pallas_call(
        _ece_kernel,
        in_specs=[
            pl.BlockSpec((128, 128), lambda: (0, 0)),
            pl.BlockSpec((128, 128), lambda: (0, 0)),
        ],
        out_specs=pl.BlockSpec((1, 1), lambda: (0, 0)),
        out_shape=jax.ShapeDtypeStruct((1, 1), jnp.float32),
    )(conf2, acc2)
    return out.reshape(1)


# speedup vs baseline: 1.0734x; 1.0734x over previous
import jax
import jax.numpy as jnp
import numpy as np
from jax.experimental import pallas as pl
from jax.experimental.pallas import tpu as pltpu

NBINS = 15
N_ROWS = 16384
N_COLS = 1000
BLOCK_ROWS = 4096
GRID = N_ROWS // BLOCK_ROWS

_BOUNDS = np.linspace(0.0, 1.0, NBINS + 1).astype(np.float32)
_LOWER = _BOUNDS[:-1].reshape(1, NBINS)
_UPPER = _BOUNDS[1:].reshape(1, NBINS)


def _ece_kernel(logits_ref, targs_ref, bounds_ref, out_ref, acc_ref):
    step = pl.program_id(0)

    @pl.when(step == 0)
    def _init():
        acc_ref[...] = jnp.zeros_like(acc_ref)

    x = logits_ref[...]
    targ = targs_ref[0, 0, :]
    col = jax.lax.broadcasted_iota(jnp.int32, x.shape, 1)
    hit = col == targ[:, None]
    m = jnp.max(x, axis=1, keepdims=True)
    x_t = jnp.max(jnp.where(hit, x, -jnp.inf), axis=1)
    s = jnp.sum(jnp.exp(x - m), axis=1)
    conf = (1.0 / s)[:, None]
    acc = (x_t == m[:, 0]).astype(jnp.float32)[:, None]

    lower = bounds_ref[0:1, :]
    upper = bounds_ref[1:2, :]
    in_bin = ((conf > lower) & (conf <= upper)).astype(jnp.float32)
    acc_ref[0, :] += jnp.sum(in_bin, axis=0)
    acc_ref[1, :] += jnp.sum(in_bin * conf, axis=0)
    acc_ref[2, :] += jnp.sum(in_bin * acc, axis=0)

    @pl.when(step == GRID - 1)
    def _fini():
        cnt = acc_ref[0, :]
        conf_sum = acc_ref[1, :]
        acc_sum = acc_ref[2, :]
        safe = jnp.maximum(cnt, 1.0)
        prop = cnt * (1.0 / N_ROWS)
        gap = jnp.abs(conf_sum / safe - acc_sum / safe) * prop
        ece = jnp.sum(jnp.where(cnt > 0.0, gap, 0.0))
        out_ref[...] = ece.reshape(1, 1)


@jax.jit
def kernel(logits, targs):
    targs3 = targs.reshape(GRID, 1, BLOCK_ROWS)
    bounds = jnp.asarray(np.concatenate([_LOWER, _UPPER], axis=0))
    out = pl.pallas_call(
        _ece_kernel,
        grid=(GRID,),
        in_specs=[
            pl.BlockSpec((BLOCK_ROWS, N_COLS), lambda i: (i, 0)),
            pl.BlockSpec((1, 1, BLOCK_ROWS), lambda i: (i, 0, 0)),
            pl.BlockSpec((2, NBINS), lambda i: (0, 0)),
        ],
        out_specs=pl.BlockSpec((1, 1), lambda i: (0, 0)),
        out_shape=jax.ShapeDtypeStruct((1, 1), jnp.float32),
        scratch_shapes=[pltpu.VMEM((3, NBINS), jnp.float32)],
    )(logits, targs3, bounds)
    return out.reshape(1)


# speedup vs baseline: 1.1297x; 1.0524x over previous
import jax
import jax.numpy as jnp
import numpy as np
from jax.experimental import pallas as pl
from jax.experimental.pallas import tpu as pltpu

NBINS = 15
N_ROWS = 16384
N_COLS = 1000
BLOCK_ROWS = 2048
N_STREAMS = 2
GRID = N_ROWS // (BLOCK_ROWS * N_STREAMS)

_BOUNDS = np.linspace(0.0, 1.0, NBINS + 1).astype(np.float32)
_LOWER = _BOUNDS[:-1].reshape(1, NBINS)
_UPPER = _BOUNDS[1:].reshape(1, NBINS)


def _ece_kernel(xa_ref, xb_ref, ta_ref, tb_ref, bounds_ref, out_ref, acc_ref):
    step = pl.program_id(0)

    @pl.when(step == 0)
    def _init():
        acc_ref[...] = jnp.zeros_like(acc_ref)

    lower = bounds_ref[0:1, :]
    upper = bounds_ref[1:2, :]
    for x_ref, t_ref in ((xa_ref, ta_ref), (xb_ref, tb_ref)):
        x = x_ref[...]
        targ = t_ref[0, 0, :]
        col = jax.lax.broadcasted_iota(jnp.int32, x.shape, 1)
        hit = col == targ[:, None]
        m = jnp.max(x, axis=1, keepdims=True)
        x_t = jnp.max(jnp.where(hit, x, -jnp.inf), axis=1)
        s = jnp.sum(jnp.exp(x - m), axis=1)
        conf = (1.0 / s)[:, None]
        acc = (x_t == m[:, 0]).astype(jnp.float32)[:, None]

        in_bin = ((conf > lower) & (conf <= upper)).astype(jnp.float32)
        acc_ref[0, :] += jnp.sum(in_bin, axis=0)
        acc_ref[1, :] += jnp.sum(in_bin * conf, axis=0)
        acc_ref[2, :] += jnp.sum(in_bin * acc, axis=0)

    @pl.when(step == GRID - 1)
    def _fini():
        cnt = acc_ref[0, :]
        conf_sum = acc_ref[1, :]
        acc_sum = acc_ref[2, :]
        safe = jnp.maximum(cnt, 1.0)
        prop = cnt * (1.0 / N_ROWS)
        gap = jnp.abs(conf_sum / safe - acc_sum / safe) * prop
        ece = jnp.sum(jnp.where(cnt > 0.0, gap, 0.0))
        out_ref[...] = ece.reshape(1, 1)


@jax.jit
def kernel(logits, targs):
    targs3 = targs.reshape(N_ROWS // BLOCK_ROWS, 1, BLOCK_ROWS)
    bounds = jnp.asarray(np.concatenate([_LOWER, _UPPER], axis=0))
    out = pl.pallas_call(
        _ece_kernel,
        grid=(GRID,),
        in_specs=[
            pl.BlockSpec((BLOCK_ROWS, N_COLS), lambda i: (2 * i, 0)),
            pl.BlockSpec((BLOCK_ROWS, N_COLS), lambda i: (2 * i + 1, 0)),
            pl.BlockSpec((1, 1, BLOCK_ROWS), lambda i: (2 * i, 0, 0)),
            pl.BlockSpec((1, 1, BLOCK_ROWS), lambda i: (2 * i + 1, 0, 0)),
            pl.BlockSpec((2, NBINS), lambda i: (0, 0)),
        ],
        out_specs=pl.BlockSpec((1, 1), lambda i: (0, 0)),
        out_shape=jax.ShapeDtypeStruct((1, 1), jnp.float32),
        scratch_shapes=[pltpu.VMEM((3, NBINS), jnp.float32)],
    )(logits, logits, targs3, targs3, bounds)
    return out.reshape(1)


# speedup vs baseline: 1.1304x; 1.0006x over previous
import jax
import jax.numpy as jnp
import numpy as np
from jax.experimental import pallas as pl
from jax.experimental.pallas import tpu as pltpu

NBINS = 15
N_ROWS = 16384
N_COLS = 1000
BLOCK_ROWS = 1024
N_STREAMS = 4
GRID = N_ROWS // (BLOCK_ROWS * N_STREAMS)

_BOUNDS = np.linspace(0.0, 1.0, NBINS + 1).astype(np.float32)
_LOWER = _BOUNDS[:-1].reshape(1, NBINS)
_UPPER = _BOUNDS[1:].reshape(1, NBINS)


def _ece_kernel(xa_ref, xb_ref, xc_ref, xd_ref, ta_ref, tb_ref, tc_ref, td_ref,
                bounds_ref, out_ref, acc_ref):
    step = pl.program_id(0)

    @pl.when(step == 0)
    def _init():
        acc_ref[...] = jnp.zeros_like(acc_ref)

    lower = bounds_ref[0:1, :]
    upper = bounds_ref[1:2, :]
    for x_ref, t_ref in ((xa_ref, ta_ref), (xb_ref, tb_ref),
                         (xc_ref, tc_ref), (xd_ref, td_ref)):
        x = x_ref[...]
        targ = t_ref[0, 0, :]
        col = jax.lax.broadcasted_iota(jnp.int32, x.shape, 1)
        hit = col == targ[:, None]
        m = jnp.max(x, axis=1, keepdims=True)
        x_t = jnp.max(jnp.where(hit, x, -jnp.inf), axis=1)
        s = jnp.sum(jnp.exp(x - m), axis=1)
        conf = (1.0 / s)[:, None]
        acc = (x_t == m[:, 0]).astype(jnp.float32)[:, None]

        in_bin = ((conf > lower) & (conf <= upper)).astype(jnp.float32)
        acc_ref[0, :] += jnp.sum(in_bin, axis=0)
        acc_ref[1, :] += jnp.sum(in_bin * conf, axis=0)
        acc_ref[2, :] += jnp.sum(in_bin * acc, axis=0)

    @pl.when(step == GRID - 1)
    def _fini():
        cnt = acc_ref[0, :]
        conf_sum = acc_ref[1, :]
        acc_sum = acc_ref[2, :]
        safe = jnp.maximum(cnt, 1.0)
        prop = cnt * (1.0 / N_ROWS)
        gap = jnp.abs(conf_sum / safe - acc_sum / safe) * prop
        ece = jnp.sum(jnp.where(cnt > 0.0, gap, 0.0))
        out_ref[...] = ece.reshape(1, 1)


@jax.jit
def kernel(logits, targs):
    targs3 = targs.reshape(N_ROWS // BLOCK_ROWS, 1, BLOCK_ROWS)
    bounds = jnp.asarray(np.concatenate([_LOWER, _UPPER], axis=0))
    out = pl.pallas_call(
        _ece_kernel,
        grid=(GRID,),
        in_specs=[
            pl.BlockSpec((BLOCK_ROWS, N_COLS), lambda i: (4 * i, 0)),
            pl.BlockSpec((BLOCK_ROWS, N_COLS), lambda i: (4 * i + 1, 0)),
            pl.BlockSpec((BLOCK_ROWS, N_COLS), lambda i: (4 * i + 2, 0)),
            pl.BlockSpec((BLOCK_ROWS, N_COLS), lambda i: (4 * i + 3, 0)),
            pl.BlockSpec((1, 1, BLOCK_ROWS), lambda i: (4 * i, 0, 0)),
            pl.BlockSpec((1, 1, BLOCK_ROWS), lambda i: (4 * i + 1, 0, 0)),
            pl.BlockSpec((1, 1, BLOCK_ROWS), lambda i: (4 * i + 2, 0, 0)),
            pl.BlockSpec((1, 1, BLOCK_ROWS), lambda i: (4 * i + 3, 0, 0)),
            pl.BlockSpec((2, NBINS), lambda i: (0, 0)),
        ],
        out_specs=pl.BlockSpec((1, 1), lambda i: (0, 0)),
        out_shape=jax.ShapeDtypeStruct((1, 1), jnp.float32),
        scratch_shapes=[pltpu.VMEM((3, NBINS), jnp.float32)],
    )(logits, logits, logits, logits, targs3, targs3, targs3, targs3, bounds)
    return out.reshape(1)
